# baseline (speedup 1.0000x reference)
import math

import jax
import jax.numpy as jnp
from jax.experimental import pallas as pl

_BLK = 2048
_HALF_LOG_2PI = 0.5 * math.log(2.0 * math.pi)


def _body(betas_ref, gumbel_ref, eps_ref, muss_ref, stdss_ref, acts_ref, lp_ref):
    blk, kk, aa = muss_ref.shape

    betas = betas_ref[...]
    logits = jnp.log(betas / jnp.sum(betas, axis=-1, keepdims=True))
    score = logits + gumbel_ref[...]

    smax = jnp.max(score, axis=-1, keepdims=True)
    kiota = jax.lax.broadcasted_iota(jnp.int32, score.shape, 1)
    comp = jnp.min(jnp.where(score == smax, kiota, kk), axis=-1, keepdims=True)

    mus = muss_ref[...]
    stds = stdss_ref[...]
    eps = eps_ref[...]

    cand = mus + stds * eps[:, None, :]
    kiota3 = jax.lax.broadcasted_iota(jnp.int32, cand.shape, 1)
    sel = jnp.sum(jnp.where(kiota3 == comp[:, :, None], cand, 0.0), axis=1)
    acts = jnp.clip(sel, -1.0, 1.0)
    acts_ref[...] = acts

    z = (acts[:, None, :] - mus) / stds
    x_terms = -0.5 * z * z - jnp.log(stds)
    log_comp = jnp.sum(x_terms, axis=-1)
    x = logits + log_comp - (aa * _HALF_LOG_2PI)
    m = jnp.max(x, axis=-1, keepdims=True)
    lp = jnp.log(jnp.sum(jnp.exp(x - m), axis=-1, keepdims=True)) + m
    lp_ref[...] = lp


def kernel(muss, stdss, betas):
    b, k, a = muss.shape
    kc, kn = jax.random.split(jax.random.key(42))
    gumbel = jax.random.gumbel(kc, (b, k), muss.dtype)
    eps = jax.random.normal(kn, (b, a), muss.dtype)

    grid = (b // _BLK,)
    acts, lp = pl.pallas_call(
        _body,
        grid=grid,
        in_specs=[
            pl.BlockSpec((_BLK, k), lambda i: (i, 0)),
            pl.BlockSpec((_BLK, k), lambda i: (i, 0)),
            pl.BlockSpec((_BLK, a), lambda i: (i, 0)),
            pl.BlockSpec((_BLK, k, a), lambda i: (i, 0, 0)),
            pl.BlockSpec((_BLK, k, a), lambda i: (i, 0, 0)),
        ],
        out_specs=[
            pl.BlockSpec((_BLK, a), lambda i: (i, 0)),
            pl.BlockSpec((_BLK, 1), lambda i: (i, 0)),
        ],
        out_shape=[
            jax.ShapeDtypeStruct((b, a), muss.dtype),
            jax.ShapeDtypeStruct((b, 1), muss.dtype),
        ],
    )(betas, gumbel, eps, muss, stdss)
    return acts, lp.reshape(b)


# speedup vs baseline: 1.0442x; 1.0442x over previous
import math

import jax
import jax.numpy as jnp
import numpy as np
from jax.experimental import pallas as pl

_BLK = 1024
_HALF_LOG_2PI = 0.5 * math.log(2.0 * math.pi)

_KN_HI, _KN_LO = 64467757, 2916123636

_ROTS = ((13, 15, 26, 6), (17, 29, 16, 24), (13, 15, 26, 6),
         (17, 29, 16, 24), (13, 15, 26, 6))
_SQRT2 = np.float32(np.sqrt(2.0))
_U_LO = np.float32(np.nextafter(np.float32(-1.0), np.float32(0.0)))


def _rotl(x, d):
    return (x << jnp.uint32(d)) | (x >> jnp.uint32(32 - d))


def _threefry_eps(base, shape):
    ks0 = jnp.uint32(_KN_HI)
    ks1 = jnp.uint32(_KN_LO)
    ks2 = ks0 ^ ks1 ^ jnp.uint32(0x1BD11BDA)
    ks = (ks0, ks1, ks2)

    j = jax.lax.broadcasted_iota(jnp.int32, shape, 0) * shape[1] \
        + jax.lax.broadcasted_iota(jnp.int32, shape, 1) + base
    x1 = j.astype(jnp.uint32) + ks1
    x0 = jnp.full(shape, ks0, jnp.uint32)
    for g, rots in enumerate(_ROTS, start=1):
        for r in rots:
            x0 = x0 + x1
            x1 = _rotl(x1, r)
            x1 = x0 ^ x1
        x0 = x0 + ks[g % 3]
        x1 = x1 + ks[(g + 1) % 3] + jnp.uint32(g)
    bits = x0 ^ x1

    fb = (bits >> jnp.uint32(9)) | jnp.uint32(0x3F800000)
    f = jax.lax.bitcast_convert_type(fb, jnp.float32) - jnp.float32(1.0)
    u = jnp.maximum(_U_LO, f * (jnp.float32(1.0) - _U_LO) + _U_LO)
    return _SQRT2 * jax.lax.erf_inv(u)


def _body(betas_ref, gumbel_ref, muss_ref, stdss_ref, acts_ref, lp_ref):
    blk, kk, aa = muss_ref.shape

    betas = betas_ref[...]
    logits = jnp.log(betas / jnp.sum(betas, axis=-1, keepdims=True))
    score = logits + gumbel_ref[...]

    smax = jnp.max(score, axis=-1, keepdims=True)
    kiota = jax.lax.broadcasted_iota(jnp.int32, score.shape, 1)
    comp = jnp.min(jnp.where(score == smax, kiota, kk), axis=-1, keepdims=True)

    eps = _threefry_eps(pl.program_id(0) * (blk * aa), (blk, aa))

    mus = muss_ref[...]
    stds = stdss_ref[...]

    cand = mus + stds * eps[:, None, :]
    kiota3 = jax.lax.broadcasted_iota(jnp.int32, cand.shape, 1)
    sel = jnp.sum(jnp.where(kiota3 == comp[:, :, None], cand, 0.0), axis=1)
    acts = jnp.clip(sel, -1.0, 1.0)
    acts_ref[...] = acts

    z = (acts[:, None, :] - mus) / stds
    x_terms = -0.5 * z * z - jnp.log(stds)
    log_comp = jnp.sum(x_terms, axis=-1)
    x = logits + log_comp - (aa * _HALF_LOG_2PI)
    m = jnp.max(x, axis=-1, keepdims=True)
    lp = jnp.log(jnp.sum(jnp.exp(x - m), axis=-1, keepdims=True)) + m
    lp_ref[...] = lp


def kernel(muss, stdss, betas):
    b, k, a = muss.shape
    kc, _ = jax.random.split(jax.random.key(42))
    gumbel = jax.random.gumbel(kc, (b, k), muss.dtype)

    grid = (b // _BLK,)
    acts, lp = pl.pallas_call(
        _body,
        grid=grid,
        in_specs=[
            pl.BlockSpec((_BLK, k), lambda i: (i, 0)),
            pl.BlockSpec((_BLK, k), lambda i: (i, 0)),
            pl.BlockSpec((_BLK, k, a), lambda i: (i, 0, 0)),
            pl.BlockSpec((_BLK, k, a), lambda i: (i, 0, 0)),
        ],
        out_specs=[
            pl.BlockSpec((_BLK, a), lambda i: (i, 0)),
            pl.BlockSpec((_BLK, 1), lambda i: (i, 0)),
        ],
        out_shape=[
            jax.ShapeDtypeStruct((b, a), muss.dtype),
            jax.ShapeDtypeStruct((b, 1), muss.dtype),
        ],
    )(betas, gumbel, muss, stdss)
    return acts, lp.reshape(b)
